# baseline (speedup 1.0000x reference)
import functools

import jax
import jax.numpy as jnp
from jax import lax
from jax.experimental import pallas as pl
from jax.experimental.pallas import tpu as pltpu
from jax.experimental.pallas import tpu_sc as plsc

_ROWS = 8192
_COLS = 512
_BLK = 256


def _sel_matrix(src_of_lane):
    import numpy as np

    e = np.zeros((128, 128), dtype=np.float32)
    for l in range(128):
        e[src_of_lane(l), l] = 1.0
    return jnp.asarray(e)


def _dot(w, e):
    return lax.dot_general(
        w, e, (((1,), (0,)), ((), ())), preferred_element_type=jnp.float32
    )


def _tc_body(x_ref, y_ref, ex0_ref, ex1_ref, ey0_ref, ey1_ref, ey2_ref,
             xo_ref, yo_ref):
    xb = x_ref[...]
    yb = y_ref[...]
    exs = (ex0_ref[...], ex1_ref[...])
    eys = (ey0_ref[...], ey1_ref[...], ey2_ref[...])
    rs = []
    for s in range(8):
        wx = xb[:, 128 * (s // 2):128 * (s // 2 + 1)]
        rs.append(_dot(wx, exs[s % 2]))
    val = jax.lax.transpose(jnp.stack(rs, axis=0), (1, 0, 2)).reshape(8 * _BLK, 128)
    xo_ref[...] = val
    for q in range(_COLS // 128):
        wy = yb[:, 128 * q:128 * (q + 1)]
        for p in range(3):
            yo_ref[:, 384 * q + 128 * p:384 * q + 128 * (p + 1)] = _dot(wy, eys[p])


@functools.cache
def _z_gather_kernel():
    mesh = plsc.VectorSubcoreMesh(core_axis_name="c", subcore_axis_name="s")

    @functools.partial(
        pl.kernel,
        mesh=mesh,
        out_type=jax.ShapeDtypeStruct((6, _COLS), jnp.float32),
        scratch_types=[
            pltpu.VMEM((8,), jnp.int32),
            pltpu.VMEM((8, _COLS), jnp.float32),
            pltpu.SemaphoreType.DMA,
        ],
    )
    def _z_gather(idx_hbm, z_hbm, out_hbm, idx_v, rows_v, sem):
        c = lax.axis_index("c")
        s = lax.axis_index("s")

        @pl.when(jnp.logical_and(c == 0, s == 0))
        def _():
            pltpu.sync_copy(idx_hbm, idx_v)
            pltpu.async_copy(z_hbm.at[idx_v], rows_v, sem).wait()
            pltpu.sync_copy(rows_v.at[pl.ds(0, 6)], out_hbm)

    return _z_gather


def kernel(x, y, z):
    zo = _z_gather_kernel()(jnp.array([0, 0, 1, 2, 2, 2, 0, 0], dtype=jnp.int32), z)

    ex0 = _sel_matrix(lambda l: l // 2)
    ex1 = _sel_matrix(lambda l: 64 + l // 2)
    ey0 = _sel_matrix(lambda l: l // 3)
    ey1 = _sel_matrix(lambda l: (128 + l) // 3)
    ey2 = _sel_matrix(lambda l: (256 + l) // 3)

    const_spec = pl.BlockSpec((128, 128), lambda i: (0, 0))
    xo, yo = pl.pallas_call(
        _tc_body,
        grid=(_ROWS // _BLK,),
        in_specs=[
            pl.BlockSpec((_BLK, _COLS), lambda i: (i, 0)),
            pl.BlockSpec((_BLK, _COLS), lambda i: (i, 0)),
            const_spec, const_spec, const_spec, const_spec, const_spec,
        ],
        out_specs=[
            pl.BlockSpec((8 * _BLK, 128), lambda i: (i, 0)),
            pl.BlockSpec((_BLK, 3 * _COLS), lambda i: (i, 0)),
        ],
        out_shape=[
            jax.ShapeDtypeStruct((8 * _ROWS, 128), jnp.float32),
            jax.ShapeDtypeStruct((_ROWS, 3 * _COLS), jnp.float32),
        ],
    )(x, y, ex0, ex1, ey0, ey1, ey2)
    return (xo.reshape(-1), yo, zo)


# speedup vs baseline: 1.2117x; 1.2117x over previous
import functools

import jax
import jax.numpy as jnp
from jax import lax
from jax.experimental import pallas as pl
from jax.experimental.pallas import tpu as pltpu
from jax.experimental.pallas import tpu_sc as plsc

_ROWS = 8192
_COLS = 512
_BLK = 1024


def _sel_matrix(src_of_lane):
    import numpy as np

    e = np.zeros((128, 128), dtype=np.float32)
    for l in range(128):
        e[src_of_lane(l), l] = 1.0
    return jnp.asarray(e)


def _dot(w, e):
    return lax.dot_general(
        w, e, (((1,), (0,)), ((), ())), preferred_element_type=jnp.float32
    )


def _tc_body(x_ref, y_ref, ex0_ref, ex1_ref, ey0_ref, ey1_ref, ey2_ref,
             xo_ref, yo_ref):
    xb = x_ref[...]
    yb = y_ref[...]
    exs = (ex0_ref[...], ex1_ref[...])
    eys = (ey0_ref[...], ey1_ref[...], ey2_ref[...])
    rs = []
    for s in range(8):
        wx = xb[:, 128 * (s // 2):128 * (s // 2 + 1)]
        rs.append(_dot(wx, exs[s % 2]))
    val = jax.lax.transpose(jnp.stack(rs, axis=0), (1, 0, 2)).reshape(8 * _BLK, 128)
    xo_ref[...] = val
    for q in range(_COLS // 128):
        wy = yb[:, 128 * q:128 * (q + 1)]
        for p in range(3):
            yo_ref[:, 384 * q + 128 * p:384 * q + 128 * (p + 1)] = _dot(wy, eys[p])


@functools.cache
def _z_gather_kernel():
    mesh = plsc.VectorSubcoreMesh(core_axis_name="c", subcore_axis_name="s")

    @functools.partial(
        pl.kernel,
        mesh=mesh,
        out_type=jax.ShapeDtypeStruct((6, _COLS), jnp.float32),
        scratch_types=[
            pltpu.VMEM((8,), jnp.int32),
            pltpu.VMEM((8, _COLS), jnp.float32),
            pltpu.SemaphoreType.DMA,
        ],
    )
    def _z_gather(idx_hbm, z_hbm, out_hbm, idx_v, rows_v, sem):
        c = lax.axis_index("c")
        s = lax.axis_index("s")

        @pl.when(jnp.logical_and(c == 0, s == 0))
        def _():
            pltpu.sync_copy(idx_hbm, idx_v)
            pltpu.async_copy(z_hbm.at[idx_v], rows_v, sem).wait()
            pltpu.sync_copy(rows_v.at[pl.ds(0, 6)], out_hbm)

    return _z_gather


def kernel(x, y, z):
    zo = _z_gather_kernel()(jnp.array([0, 0, 1, 2, 2, 2, 0, 0], dtype=jnp.int32), z)

    ex0 = _sel_matrix(lambda l: l // 2)
    ex1 = _sel_matrix(lambda l: 64 + l // 2)
    ey0 = _sel_matrix(lambda l: l // 3)
    ey1 = _sel_matrix(lambda l: (128 + l) // 3)
    ey2 = _sel_matrix(lambda l: (256 + l) // 3)

    const_spec = pl.BlockSpec((128, 128), lambda i: (0, 0))
    xo, yo = pl.pallas_call(
        _tc_body,
        grid=(_ROWS // _BLK,),
        in_specs=[
            pl.BlockSpec((_BLK, _COLS), lambda i: (i, 0)),
            pl.BlockSpec((_BLK, _COLS), lambda i: (i, 0)),
            const_spec, const_spec, const_spec, const_spec, const_spec,
        ],
        out_specs=[
            pl.BlockSpec((8 * _BLK, 128), lambda i: (i, 0)),
            pl.BlockSpec((_BLK, 3 * _COLS), lambda i: (i, 0)),
        ],
        out_shape=[
            jax.ShapeDtypeStruct((8 * _ROWS, 128), jnp.float32),
            jax.ShapeDtypeStruct((_ROWS, 3 * _COLS), jnp.float32),
        ],
    )(x, y, ex0, ex1, ey0, ey1, ey2)
    return (xo.reshape(-1), yo, zo)
